# baseline (speedup 1.0000x reference)
import functools

import jax
import jax.numpy as jnp
from jax.experimental import pallas as pl
from jax.experimental.pallas import tpu as pltpu

PACK_W = 128
_MAX_TB = 1024


def _fused_kernel(xl_ref, xn_ref, act_ref, aw1_ref, ab1_ref, cw1_ref, cb1_ref,
                  w2a_ref, w2c_ref, b2_ref, probs_ref, scalT_ref, v_ref, *,
                  num_actions):
    A = num_actions

    bf = jnp.bfloat16
    h_a = jnp.dot(xl_ref[...], aw1_ref[...].astype(bf),
                  preferred_element_type=jnp.float32) + ab1_ref[...]
    h_a = jnp.maximum(h_a, 0.0)
    h_c = jnp.dot(xn_ref[...].astype(bf), cw1_ref[...].astype(bf),
                  preferred_element_type=jnp.float32) + cb1_ref[...]
    h_c = jnp.maximum(h_c, 0.0)

    z = (jnp.dot(h_a.astype(bf), w2a_ref[...].astype(bf),
                 preferred_element_type=jnp.float32)
         + b2_ref[...]
         + jnp.dot(h_c.astype(bf), w2c_ref[...].astype(bf),
                   preferred_element_type=jnp.float32))

    cols = jax.lax.broadcasted_iota(jnp.int32, z.shape, 1)
    is_logit = cols < A
    logits = jnp.where(is_logit, z, jnp.float32(-1e30))
    m = jnp.max(logits, axis=-1, keepdims=True)
    shifted = logits - m
    e = jnp.exp(shifted)
    denom = jnp.sum(e, axis=-1, keepdims=True)
    log_softmax = shifted - jnp.log(denom)
    probs = e / denom

    ent = -jnp.sum(jnp.where(is_logit, probs * log_softmax, 0.0),
                   axis=-1, keepdims=True)

    a = act_ref[...]
    logp = jnp.sum(jnp.where(cols == a, log_softmax, 0.0),
                   axis=-1, keepdims=True)

    v = jnp.sum(jnp.where(cols == A, z, 0.0), axis=-1, keepdims=True)

    probs_ref[...] = probs[:, :probs_ref.shape[1]]

    scal = jnp.concatenate(
        [logp, ent, v, jnp.zeros((logp.shape[0], 5), jnp.float32)], axis=1)
    scalT_ref[...] = scal.T
    v_ref[...] = v


def _blkspec(shape, idx):
    return pl.BlockSpec(shape, lambda i: idx,
                        memory_space=pltpu.MemorySpace.VMEM)


def _bspec(tb, width):
    return pl.BlockSpec((tb, width), lambda i: (i, 0),
                        memory_space=pltpu.MemorySpace.VMEM)


def _round_up(n, m):
    return ((n + m - 1) // m) * m


def _evaluate_pallas(xl, xn, act, a_w1, a_b1, w1, b1, w2, b2, A, b_pad, tb):
    dl = xl.shape[1]
    dn = xn.shape[1]
    h = a_w1.shape[1]
    probs, scalT, vout = pl.pallas_call(
        functools.partial(_fused_kernel, num_actions=A),
        out_shape=(jax.ShapeDtypeStruct((b_pad, A), jnp.float32),
                   jax.ShapeDtypeStruct((8, b_pad), jnp.float32),
                   jax.ShapeDtypeStruct((b_pad, 1), jnp.float32)),
        grid_spec=pltpu.PrefetchScalarGridSpec(
            num_scalar_prefetch=0,
            grid=(b_pad // tb,),
            in_specs=[_bspec(tb, dl),
                      _bspec(tb, dn),
                      _bspec(tb, 1),
                      _blkspec(a_w1.shape, (0, 0)),
                      _blkspec(a_b1.shape, (0, 0)),
                      _blkspec((dn, h), (1, 1)),
                      _blkspec((1, h), (0, 1)),
                      _blkspec((h, PACK_W), (0, 0)),
                      _blkspec((h, PACK_W), (1, 0)),
                      _blkspec(b2.shape, (0, 0))],
            out_specs=(_bspec(tb, A),
                       pl.BlockSpec((8, tb), lambda i: (0, i),
                                    memory_space=pltpu.MemorySpace.VMEM),
                       _bspec(tb, 1))),
        compiler_params=pltpu.CompilerParams(
            dimension_semantics=("parallel",)),
    )(xl, xn, act, a_w1, a_b1, w1, b1, w2, w2, b2)
    return probs, scalT, vout


@jax.jit
def kernel(neural_state, logic_state, action, w1, b1, w2, b2,
           a_w1, a_b1, a_w2, a_b2):
    B = logic_state.shape[0]
    A = a_w2.shape[1]

    xl = logic_state.reshape(B, -1).astype(jnp.bfloat16)
    xn = neural_state.reshape(B, -1).astype(jnp.float32)
    act = action.astype(jnp.int32).reshape(B, 1)

    if B <= _MAX_TB:
        b_pad = _round_up(B, 8)
        tb = b_pad
    else:
        b_pad = _round_up(B, _MAX_TB)
        tb = _MAX_TB
    if b_pad != B:
        xl = jnp.pad(xl, ((0, b_pad - B), (0, 0)))
        xn = jnp.pad(xn, ((0, b_pad - B), (0, 0)))
        act = jnp.pad(act, ((0, b_pad - B), (0, 0)))

    probs, scalT, vout = _evaluate_pallas(xl, xn, act, a_w1, a_b1, w1, b1,
                                          w2, b2, A, b_pad, tb)

    action_probs = probs if b_pad == B else probs[:B]
    action_logprobs = scalT[0, :B]
    dist_entropy = scalT[1, :B]
    state_values = vout if b_pad == B else vout[:B]
    return action_logprobs, state_values, dist_entropy, action_probs


# speedup vs baseline: 1.0588x; 1.0588x over previous
import functools

import jax
import jax.numpy as jnp
from jax.experimental import pallas as pl
from jax.experimental.pallas import tpu as pltpu

PACK_W = 128
_MAX_TB = 1024


def _fused_kernel(xl_ref, xn_ref, act_ref, aw1_ref, ab1_ref, cw1_ref, cb1_ref,
                  w2a_ref, w2c_ref, b2_ref, probs_ref, scalT_ref, *,
                  num_actions):
    A = num_actions

    bf = jnp.bfloat16
    h_a = jnp.dot(xl_ref[...], aw1_ref[...].astype(bf),
                  preferred_element_type=jnp.float32) + ab1_ref[...]
    h_a = jnp.maximum(h_a, 0.0)
    h_c = jnp.dot(xn_ref[...].astype(bf), cw1_ref[...].astype(bf),
                  preferred_element_type=jnp.float32) + cb1_ref[...]
    h_c = jnp.maximum(h_c, 0.0)

    z = (jnp.dot(h_a.astype(bf), w2a_ref[...].astype(bf),
                 preferred_element_type=jnp.float32)
         + b2_ref[...]
         + jnp.dot(h_c.astype(bf), w2c_ref[...].astype(bf),
                   preferred_element_type=jnp.float32))

    cols = jax.lax.broadcasted_iota(jnp.int32, z.shape, 1)
    is_logit = cols < A
    logits = jnp.where(is_logit, z, jnp.float32(-1e30))
    m = jnp.max(logits, axis=-1, keepdims=True)
    shifted = logits - m
    e = jnp.exp(shifted)
    denom = jnp.sum(e, axis=-1, keepdims=True)
    log_softmax = shifted - jnp.log(denom)
    probs = e / denom

    ent = -jnp.sum(probs * log_softmax, axis=-1, keepdims=True)

    a = act_ref[...]
    logp = jnp.sum(jnp.where(cols == a, log_softmax, 0.0),
                   axis=-1, keepdims=True)

    v = jnp.sum(jnp.where(cols == A, z, 0.0), axis=-1, keepdims=True)

    probs_ref[...] = probs[:, :probs_ref.shape[1]]

    scal = jnp.concatenate(
        [logp, ent, v, jnp.zeros((logp.shape[0], 5), jnp.float32)], axis=1)
    scalT_ref[...] = scal.T


def _blkspec(shape, idx):
    return pl.BlockSpec(shape, lambda i: idx,
                        memory_space=pltpu.MemorySpace.VMEM)


def _bspec(tb, width):
    return pl.BlockSpec((tb, width), lambda i: (i, 0),
                        memory_space=pltpu.MemorySpace.VMEM)


def _round_up(n, m):
    return ((n + m - 1) // m) * m


def _evaluate_pallas(xl, xn, act, a_w1, a_b1, w1, b1, w2, b2, A, b_pad, tb):
    dl = xl.shape[1]
    dn = xn.shape[1]
    h = a_w1.shape[1]
    probs, scalT = pl.pallas_call(
        functools.partial(_fused_kernel, num_actions=A),
        out_shape=(jax.ShapeDtypeStruct((b_pad, A), jnp.float32),
                   jax.ShapeDtypeStruct((8, b_pad), jnp.float32)),
        grid_spec=pltpu.PrefetchScalarGridSpec(
            num_scalar_prefetch=0,
            grid=(b_pad // tb,),
            in_specs=[_bspec(tb, dl),
                      _bspec(tb, dn),
                      _bspec(tb, 1),
                      _blkspec(a_w1.shape, (0, 0)),
                      _blkspec(a_b1.shape, (0, 0)),
                      _blkspec((dn, h), (1, 1)),
                      _blkspec((1, h), (0, 1)),
                      _blkspec((h, PACK_W), (0, 0)),
                      _blkspec((h, PACK_W), (1, 0)),
                      _blkspec(b2.shape, (0, 0))],
            out_specs=(_bspec(tb, A),
                       pl.BlockSpec((8, tb), lambda i: (0, i),
                                    memory_space=pltpu.MemorySpace.VMEM))),
        compiler_params=pltpu.CompilerParams(
            dimension_semantics=("parallel",)),
    )(xl, xn, act, a_w1, a_b1, w1, b1, w2, w2, b2)
    return probs, scalT


@jax.jit
def kernel(neural_state, logic_state, action, w1, b1, w2, b2,
           a_w1, a_b1, a_w2, a_b2):
    B = logic_state.shape[0]
    A = a_w2.shape[1]

    xl = logic_state.reshape(B, -1).astype(jnp.bfloat16)
    xn = neural_state.reshape(B, -1).astype(jnp.float32)
    act = action.astype(jnp.int32).reshape(B, 1)

    if B <= _MAX_TB:
        b_pad = _round_up(B, 8)
        tb = b_pad
    else:
        b_pad = _round_up(B, _MAX_TB)
        tb = _MAX_TB
    if b_pad != B:
        xl = jnp.pad(xl, ((0, b_pad - B), (0, 0)))
        xn = jnp.pad(xn, ((0, b_pad - B), (0, 0)))
        act = jnp.pad(act, ((0, b_pad - B), (0, 0)))

    probs, scalT = _evaluate_pallas(xl, xn, act, a_w1, a_b1, w1, b1, w2, b2,
                                    A, b_pad, tb)

    action_probs = probs if b_pad == B else probs[:B]
    action_logprobs = scalT[0, :B]
    dist_entropy = scalT[1, :B]
    state_values = scalT[2, :B].reshape(B, 1)
    return action_logprobs, state_values, dist_entropy, action_probs


# speedup vs baseline: 1.0735x; 1.0139x over previous
import functools

import jax
import jax.numpy as jnp
from jax.experimental import pallas as pl
from jax.experimental.pallas import tpu as pltpu

PACK_W = 128
_MAX_TB = 2048


def _fused_kernel(xl_ref, xn_ref, act_ref, aw1_ref, ab1_ref, cw1_ref, cb1_ref,
                  w2a_ref, w2c_ref, b2_ref, probs_ref, scalT_ref, *,
                  num_actions):
    A = num_actions

    bf = jnp.bfloat16
    h_a = jnp.dot(xl_ref[...], aw1_ref[...].astype(bf),
                  preferred_element_type=jnp.float32) + ab1_ref[...]
    h_a = jnp.maximum(h_a, 0.0)
    h_c = jnp.dot(xn_ref[...].astype(bf), cw1_ref[...].astype(bf),
                  preferred_element_type=jnp.float32) + cb1_ref[...]
    h_c = jnp.maximum(h_c, 0.0)

    z = (jnp.dot(h_a.astype(bf), w2a_ref[...].astype(bf),
                 preferred_element_type=jnp.float32)
         + b2_ref[...]
         + jnp.dot(h_c.astype(bf), w2c_ref[...].astype(bf),
                   preferred_element_type=jnp.float32))

    cols = jax.lax.broadcasted_iota(jnp.int32, z.shape, 1)
    is_logit = cols < A
    logits = jnp.where(is_logit, z, jnp.float32(-1e30))
    m = jnp.max(logits, axis=-1, keepdims=True)
    shifted = logits - m
    e = jnp.exp(shifted)
    denom = jnp.sum(e, axis=-1, keepdims=True)
    log_softmax = shifted - jnp.log(denom)
    probs = e / denom

    ent = -jnp.sum(probs * log_softmax, axis=-1, keepdims=True)

    a = act_ref[...]
    logp = jnp.sum(jnp.where(cols == a, log_softmax, 0.0),
                   axis=-1, keepdims=True)

    v = jnp.sum(jnp.where(cols == A, z, 0.0), axis=-1, keepdims=True)

    probs_ref[...] = probs[:, :probs_ref.shape[1]]

    scal = jnp.concatenate(
        [logp, ent, v, jnp.zeros((logp.shape[0], 5), jnp.float32)], axis=1)
    scalT_ref[...] = scal.T


def _blkspec(shape, idx):
    return pl.BlockSpec(shape, lambda i: idx,
                        memory_space=pltpu.MemorySpace.VMEM)


def _bspec(tb, width):
    return pl.BlockSpec((tb, width), lambda i: (i, 0),
                        memory_space=pltpu.MemorySpace.VMEM)


def _round_up(n, m):
    return ((n + m - 1) // m) * m


def _evaluate_pallas(xl, xn, act, a_w1, a_b1, w1, b1, w2, b2, A, b_pad, tb):
    dl = xl.shape[1]
    dn = xn.shape[1]
    h = a_w1.shape[1]
    probs, scalT = pl.pallas_call(
        functools.partial(_fused_kernel, num_actions=A),
        out_shape=(jax.ShapeDtypeStruct((b_pad, A), jnp.float32),
                   jax.ShapeDtypeStruct((8, b_pad), jnp.float32)),
        grid_spec=pltpu.PrefetchScalarGridSpec(
            num_scalar_prefetch=0,
            grid=(b_pad // tb,),
            in_specs=[_bspec(tb, dl),
                      _bspec(tb, dn),
                      _bspec(tb, 1),
                      _blkspec(a_w1.shape, (0, 0)),
                      _blkspec(a_b1.shape, (0, 0)),
                      _blkspec((dn, h), (1, 1)),
                      _blkspec((1, h), (0, 1)),
                      _blkspec((h, PACK_W), (0, 0)),
                      _blkspec((h, PACK_W), (1, 0)),
                      _blkspec(b2.shape, (0, 0))],
            out_specs=(_bspec(tb, A),
                       pl.BlockSpec((8, tb), lambda i: (0, i),
                                    memory_space=pltpu.MemorySpace.VMEM))),
        compiler_params=pltpu.CompilerParams(
            dimension_semantics=("parallel",)),
    )(xl, xn, act, a_w1, a_b1, w1, b1, w2, w2, b2)
    return probs, scalT


@jax.jit
def kernel(neural_state, logic_state, action, w1, b1, w2, b2,
           a_w1, a_b1, a_w2, a_b2):
    B = logic_state.shape[0]
    A = a_w2.shape[1]

    xl = logic_state.reshape(B, -1).astype(jnp.bfloat16)
    xn = neural_state.reshape(B, -1).astype(jnp.float32)
    act = action.astype(jnp.int32).reshape(B, 1)

    if B <= _MAX_TB:
        b_pad = _round_up(B, 8)
        tb = b_pad
    else:
        b_pad = _round_up(B, _MAX_TB)
        tb = _MAX_TB
    if b_pad != B:
        xl = jnp.pad(xl, ((0, b_pad - B), (0, 0)))
        xn = jnp.pad(xn, ((0, b_pad - B), (0, 0)))
        act = jnp.pad(act, ((0, b_pad - B), (0, 0)))

    probs, scalT = _evaluate_pallas(xl, xn, act, a_w1, a_b1, w1, b1, w2, b2,
                                    A, b_pad, tb)

    action_probs = probs if b_pad == B else probs[:B]
    action_logprobs = scalT[0, :B]
    dist_entropy = scalT[1, :B]
    state_values = scalT[2, :B].reshape(B, 1)
    return action_logprobs, state_values, dist_entropy, action_probs
